# baseline (speedup 1.0000x reference)
import functools

import jax
import jax.numpy as jnp
from jax import lax
from jax.experimental import pallas as pl
from jax.experimental.pallas import tpu as pltpu
from jax.experimental.pallas import tpu_sc as plsc

_NBUF = 7
_LEAD = 4


def _scramble_rows(x0f, idx2, n_rows, d, n_workers, rows_per_worker, chunk):
    n_chunks = rows_per_worker // chunk
    n_loop = (n_chunks // _NBUF) * _NBUF
    mesh = plsc.VectorSubcoreMesh(core_axis_name="c", subcore_axis_name="s")

    @functools.partial(
        pl.kernel,
        mesh=mesh,
        out_type=jax.ShapeDtypeStruct((n_rows, d), jnp.float32),
        scratch_types=[
            pltpu.VMEM((rows_per_worker,), jnp.int32),
            pltpu.VMEM((_NBUF, chunk, d), jnp.float32),
        ]
        + [pltpu.SemaphoreType.DMA] * (2 * _NBUF),
    )
    def body(x0_hbm, idx_hbm, out_hbm, idx_v, bufs, *sems):
        gsems = sems[:_NBUF]
        wsems = sems[_NBUF:]
        wid = lax.axis_index("s") * 2 + lax.axis_index("c")
        base = wid * rows_per_worker
        pltpu.sync_copy(idx_hbm.at[wid], idx_v)

        def fire_g(ch, slot):
            pltpu.async_copy(
                x0_hbm.at[idx_v.at[pl.ds(ch * chunk, chunk)]],
                bufs.at[slot],
                gsems[slot],
            )

        def wait_g(slot):
            pltpu.make_async_copy(
                x0_hbm.at[idx_v.at[pl.ds(0, chunk)]],
                bufs.at[slot],
                gsems[slot],
            ).wait()

        def fire_w(ch, slot):
            pltpu.async_copy(
                bufs.at[slot],
                out_hbm.at[pl.ds(base + ch * chunk, chunk)],
                wsems[slot],
            )

        def wait_w(slot):
            pltpu.make_async_copy(
                bufs.at[slot],
                out_hbm.at[pl.ds(base, chunk)],
                wsems[slot],
            ).wait()

        for p in range(_LEAD):
            fire_g(p, p)

        def visit(ch, bb):
            wait_g(bb)
            fire_w(ch, bb)
            b2 = (bb + _LEAD) % _NBUF

            @pl.when(jnp.logical_and(ch >= _NBUF - _LEAD, ch + _LEAD < n_chunks))
            def _():
                wait_w(b2)
                fire_g(ch + _LEAD, b2)

            @pl.when(jnp.logical_and(ch < _NBUF - _LEAD, ch + _LEAD < n_chunks))
            def _():
                fire_g(ch + _LEAD, b2)

        def step(i, _):
            for bb in range(_NBUF):
                visit(_NBUF * i + bb, bb)
            return 0

        lax.fori_loop(0, n_loop // _NBUF, step, 0)
        for ch in range(n_loop, n_chunks):
            bb = ch % _NBUF
            wait_g(bb)
            fire_w(ch, bb)
        for bb in range(_NBUF):
            wait_w(bb)

    return body(x0f, idx2)


def kernel(x0, x1, x2, perm):
    b, s, d = x0.shape
    n = b * s
    n_workers = 32
    rows_per_worker = n // n_workers
    chunk = 8

    x0f = x0.reshape(n, d)
    idx = (
        jnp.arange(b, dtype=jnp.int32)[:, None] * s + perm[None, :].astype(jnp.int32)
    ).reshape(n_workers, rows_per_worker)

    outf = _scramble_rows(x0f, idx, n, d, n_workers, rows_per_worker, chunk)
    return outf.reshape(b, s, d), x1, x2


# speedup vs baseline: 1.0050x; 1.0050x over previous
import functools

import jax
import jax.numpy as jnp
from jax import lax
from jax.experimental import pallas as pl
from jax.experimental.pallas import tpu as pltpu
from jax.experimental.pallas import tpu_sc as plsc

_NBUF = 7
_LEAD = 5


def _scramble_rows(x0f, idx2, n_rows, d, n_workers, rows_per_worker, chunk):
    n_chunks = rows_per_worker // chunk
    n_loop = (n_chunks // _NBUF) * _NBUF
    mesh = plsc.VectorSubcoreMesh(core_axis_name="c", subcore_axis_name="s")

    @functools.partial(
        pl.kernel,
        mesh=mesh,
        out_type=jax.ShapeDtypeStruct((n_rows, d), jnp.float32),
        scratch_types=[
            pltpu.VMEM((rows_per_worker,), jnp.int32),
            pltpu.VMEM((_NBUF, chunk, d), jnp.float32),
        ]
        + [pltpu.SemaphoreType.DMA] * (2 * _NBUF),
    )
    def body(x0_hbm, idx_hbm, out_hbm, idx_v, bufs, *sems):
        gsems = sems[:_NBUF]
        wsems = sems[_NBUF:]
        wid = lax.axis_index("s") * 2 + lax.axis_index("c")
        base = wid * rows_per_worker
        pltpu.sync_copy(idx_hbm.at[wid], idx_v)

        def fire_g(ch, slot):
            pltpu.async_copy(
                x0_hbm.at[idx_v.at[pl.ds(ch * chunk, chunk)]],
                bufs.at[slot],
                gsems[slot],
            )

        def wait_g(slot):
            pltpu.make_async_copy(
                x0_hbm.at[idx_v.at[pl.ds(0, chunk)]],
                bufs.at[slot],
                gsems[slot],
            ).wait()

        def fire_w(ch, slot):
            pltpu.async_copy(
                bufs.at[slot],
                out_hbm.at[pl.ds(base + ch * chunk, chunk)],
                wsems[slot],
            )

        def wait_w(slot):
            pltpu.make_async_copy(
                bufs.at[slot],
                out_hbm.at[pl.ds(base, chunk)],
                wsems[slot],
            ).wait()

        for p in range(_LEAD):
            fire_g(p, p)

        def visit(ch, bb):
            wait_g(bb)
            fire_w(ch, bb)
            b2 = (bb + _LEAD) % _NBUF

            @pl.when(jnp.logical_and(ch >= _NBUF - _LEAD, ch + _LEAD < n_chunks))
            def _():
                wait_w(b2)
                fire_g(ch + _LEAD, b2)

            @pl.when(jnp.logical_and(ch < _NBUF - _LEAD, ch + _LEAD < n_chunks))
            def _():
                fire_g(ch + _LEAD, b2)

        def step(i, _):
            for bb in range(_NBUF):
                visit(_NBUF * i + bb, bb)
            return 0

        lax.fori_loop(0, n_loop // _NBUF, step, 0)
        for ch in range(n_loop, n_chunks):
            bb = ch % _NBUF
            wait_g(bb)
            fire_w(ch, bb)
        for bb in range(_NBUF):
            wait_w(bb)

    return body(x0f, idx2)


def kernel(x0, x1, x2, perm):
    b, s, d = x0.shape
    n = b * s
    n_workers = 32
    rows_per_worker = n // n_workers
    chunk = 8

    x0f = x0.reshape(n, d)
    idx = (
        jnp.arange(b, dtype=jnp.int32)[:, None] * s + perm[None, :].astype(jnp.int32)
    ).reshape(n_workers, rows_per_worker)

    outf = _scramble_rows(x0f, idx, n, d, n_workers, rows_per_worker, chunk)
    return outf.reshape(b, s, d), x1, x2
